# Initial kernel scaffold; baseline (speedup 1.0000x reference)
import jax
import jax.numpy as jnp
from jax.experimental import pallas as pl


def kernel(word_input, sub_category_input, table, W_cls, b_cls):
    raise NotImplementedError("write your pallas kernel here")


# speedup vs baseline: 1.7638x; 1.7638x over previous
import functools

import jax
import jax.numpy as jnp
from jax import lax
from jax.experimental import pallas as pl
from jax.experimental.pallas import tpu as pltpu
from jax.experimental.pallas import tpu_sc as plsc

B = 4096
L = 200
EMBED = 128
SUB = 128
NUM_CLASSES = 10000

NC, NS = 2, 16
NW = NC * NS
RPW = B // NW
CHUNK = 104
NCH = 2
LPAD = CHUNK * NCH
VEC = 16


def _sc_pool(table, idx3):
    mesh = plsc.VectorSubcoreMesh(core_axis_name="c", subcore_axis_name="s")

    @functools.partial(
        pl.kernel,
        out_type=jax.ShapeDtypeStruct((B, EMBED), jnp.float32),
        mesh=mesh,
        scratch_types=[
            pltpu.VMEM((RPW, NCH, CHUNK), jnp.int32),
            pltpu.VMEM((CHUNK, EMBED), jnp.float32),
            pltpu.VMEM((RPW, EMBED), jnp.float32),
            pltpu.SemaphoreType.DMA,
        ],
    )
    def k(table_hbm, idx_hbm, out_hbm, idx_v, gbuf, obuf, sem):
        wid = lax.axis_index("s") * NC + lax.axis_index("c")
        base = wid * RPW
        pltpu.sync_copy(idx_hbm.at[pl.ds(base, RPW)], idx_v)

        def add_row(j, acc):
            return tuple(
                acc[kk] + gbuf[j, pl.ds(VEC * kk, VEC)]
                for kk in range(EMBED // VEC)
            )

        def row_body(r, carry):
            acc = tuple(
                jnp.zeros((VEC,), jnp.float32) for _ in range(EMBED // VEC)
            )
            for c in range(NCH):
                pltpu.async_copy(table_hbm.at[idx_v.at[r, c]], gbuf, sem).wait()
                acc = lax.fori_loop(0, CHUNK, add_row, acc)
            scale = jnp.float32(1.0 / L)
            for kk in range(EMBED // VEC):
                obuf[r, pl.ds(VEC * kk, VEC)] = acc[kk] * scale
            return carry

        lax.fori_loop(0, RPW, row_body, jnp.int32(0))
        pltpu.sync_copy(obuf, out_hbm.at[pl.ds(base, RPW)])

    return k(table, idx3)


BM = 512
BN = 1024


def _mm_kernel(p_ref, s_ref, w_ref, b_ref, o_ref):
    p = p_ref[...].astype(jnp.bfloat16)
    s = s_ref[...].astype(jnp.bfloat16)
    w = w_ref[...].astype(jnp.bfloat16)
    dn = (((1,), (1,)), ((), ()))
    acc = lax.dot_general(p, w[:, :EMBED], dn,
                          preferred_element_type=jnp.float32)
    acc = acc + lax.dot_general(s, w[:, EMBED:], dn,
                                preferred_element_type=jnp.float32)
    o_ref[...] = acc + b_ref[...]


def _tc_classify(pooled, sub, W_cls, b_cls):
    grid = (B // BM, pl.cdiv(NUM_CLASSES, BN))
    return pl.pallas_call(
        _mm_kernel,
        grid=grid,
        in_specs=[
            pl.BlockSpec((BM, EMBED), lambda i, j: (i, 0)),
            pl.BlockSpec((BM, SUB), lambda i, j: (i, 0)),
            pl.BlockSpec((BN, EMBED + SUB), lambda i, j: (j, 0)),
            pl.BlockSpec((1, BN), lambda i, j: (0, j)),
        ],
        out_specs=pl.BlockSpec((BM, BN), lambda i, j: (i, j)),
        out_shape=jax.ShapeDtypeStruct((B, NUM_CLASSES), jnp.float32),
        compiler_params=pltpu.CompilerParams(
            dimension_semantics=("parallel", "parallel"),
        ),
    )(pooled, sub, W_cls, b_cls.reshape(1, NUM_CLASSES))


def kernel(word_input, sub_category_input, table, W_cls, b_cls):
    idx = word_input.astype(jnp.int32)
    idx = jnp.pad(idx, ((0, 0), (0, LPAD - L)))
    idx3 = idx.reshape(B, NCH, CHUNK)
    pooled = _sc_pool(table, idx3)
    return _tc_classify(pooled, sub_category_input, W_cls, b_cls)
